# baseline (speedup 1.0000x reference)
import functools

import jax
import jax.numpy as jnp
from jax import lax
from jax.experimental import pallas as pl
from jax.experimental.pallas import tpu as pltpu
from jax.experimental.pallas import tpu_sc as plsc

NC, NS, L = 2, 16, 16
NW = NC * NS
SHAPE = (512, 2, 300, 25)
NTOT = 512 * 2 * 300 * 25
PER_W = NTOT // NW
CHUNK = 48_000
NCHUNK = PER_W // CHUNK

_mesh = plsc.VectorSubcoreMesh(core_axis_name="c", subcore_axis_name="s")


@functools.partial(
    pl.kernel,
    out_type=(
        jax.ShapeDtypeStruct((NTOT,), jnp.float32),
        jax.ShapeDtypeStruct((NTOT,), jnp.float32),
    ),
    mesh=_mesh,
    scratch_types=[
        pltpu.VMEM((CHUNK,), jnp.float32),
        pltpu.VMEM((CHUNK,), jnp.float32),
    ],
)
def _sc_mask(m0_hbm, m1_hbm, out0_hbm, out1_hbm, a_buf, b_buf):
    wid = lax.axis_index("s") * NC + lax.axis_index("c")
    base = wid * PER_W

    @pl.loop(0, NCHUNK)
    def _chunk(i):
        off = base + i * CHUNK
        pltpu.sync_copy(m0_hbm.at[pl.ds(off, CHUNK)], a_buf)
        pltpu.sync_copy(m1_hbm.at[pl.ds(off, CHUNK)], b_buf)

        @pl.loop(0, CHUNK // L, unroll=8)
        def _vec(j):
            s = pl.ds(j * L, L)
            b_buf[s] = a_buf[s] * b_buf[s]

        pltpu.sync_copy(a_buf, out0_hbm.at[pl.ds(off, CHUNK)])
        pltpu.sync_copy(b_buf, out1_hbm.at[pl.ds(off, CHUNK)])


def kernel(m0, m1, m2):
    del m2
    out0, out1 = _sc_mask(m0.reshape(-1), m1.reshape(-1))
    return out0.reshape(SHAPE), out1.reshape(SHAPE)


# speedup vs baseline: 1.0158x; 1.0158x over previous
import functools

import jax
import jax.numpy as jnp
from jax import lax
from jax.experimental import pallas as pl
from jax.experimental.pallas import tpu as pltpu
from jax.experimental.pallas import tpu_sc as plsc

NC, NS = 2, 16
NW = NC * NS
SHAPE = (512, 2, 300, 25)
NTOT = 512 * 2 * 300 * 25
PER_W = NTOT // NW
ROWS = NTOT // 128
BLK = 4000

_mesh = plsc.VectorSubcoreMesh(core_axis_name="c", subcore_axis_name="s")


def _to_phys(x):
    return x.reshape(4, 128, 2, 300, 25).transpose(3, 4, 0, 2, 1).reshape(-1)


def _from_phys(v):
    return v.reshape(300, 25, 4, 2, 128).transpose(2, 4, 3, 0, 1).reshape(SHAPE)


@functools.partial(
    pl.kernel,
    out_type=jax.ShapeDtypeStruct((NTOT,), jnp.float32),
    mesh=_mesh,
)
def _sc_copy(src_hbm, dst_hbm):
    wid = lax.axis_index("s") * NC + lax.axis_index("c")
    off = wid * PER_W
    pltpu.sync_copy(src_hbm.at[pl.ds(off, PER_W)], dst_hbm.at[pl.ds(off, PER_W)])


def _mul_body(a_ref, b_ref, o_ref):
    o_ref[...] = a_ref[...] * b_ref[...]


_tc_mul = pl.pallas_call(
    _mul_body,
    out_shape=jax.ShapeDtypeStruct((ROWS, 128), jnp.float32),
    grid=(ROWS // BLK,),
    in_specs=[
        pl.BlockSpec((BLK, 128), lambda i: (i, 0)),
        pl.BlockSpec((BLK, 128), lambda i: (i, 0)),
    ],
    out_specs=pl.BlockSpec((BLK, 128), lambda i: (i, 0)),
)


def kernel(m0, m1, m2):
    del m2
    pm0 = _to_phys(m0)
    pm1 = _to_phys(m1)
    out0 = _from_phys(_sc_copy(pm0))
    out1 = _from_phys(_tc_mul(pm0.reshape(ROWS, 128), pm1.reshape(ROWS, 128)).reshape(-1))
    return out0, out1


# speedup vs baseline: 14.4860x; 14.2602x over previous
import functools

import jax
import jax.numpy as jnp
from jax import lax
from jax.experimental import pallas as pl
from jax.experimental.pallas import tpu as pltpu
from jax.experimental.pallas import tpu_sc as plsc

NC, NS = 2, 16
NW = NC * NS
SHAPE = (512, 2, 300, 25)
NTOT = 512 * 2 * 300 * 25
PER_W = NTOT // NW
ROWS = NTOT // 128
BLK = 4000

_mesh = plsc.VectorSubcoreMesh(core_axis_name="c", subcore_axis_name="s")


def _to_phys(x):
    return x.reshape(4, 128, 2, 300, 25).transpose(3, 4, 0, 2, 1).reshape(-1)


def _from_phys(v):
    return v.reshape(300, 25, 4, 2, 128).transpose(2, 4, 3, 0, 1).reshape(SHAPE)


CHUNK = 40_000
NCHUNK = PER_W // CHUNK
NSLOT = 3


@functools.partial(
    pl.kernel,
    out_type=jax.ShapeDtypeStruct((NTOT,), jnp.float32),
    mesh=_mesh,
    scratch_types=(
        [pltpu.VMEM((CHUNK,), jnp.float32) for _ in range(NSLOT)]
        + [pltpu.SemaphoreType.DMA for _ in range(2 * NSLOT)]
    ),
)
def _sc_copy(src_hbm, dst_hbm, b0, b1, b2, si0, si1, si2, so0, so1, so2):
    bufs = (b0, b1, b2)
    sin = (si0, si1, si2)
    sout = (so0, so1, so2)
    wid = lax.axis_index("s") * NC + lax.axis_index("c")
    base = wid * PER_W

    def start_in(i):
        off = base + i * CHUNK
        return pltpu.async_copy(src_hbm.at[pl.ds(off, CHUNK)], bufs[i % NSLOT],
                                sin[i % NSLOT])

    def start_out(i):
        off = base + i * CHUNK
        return pltpu.async_copy(bufs[i % NSLOT], dst_hbm.at[pl.ds(off, CHUNK)],
                                sout[i % NSLOT])

    din = {i: start_in(i) for i in range(NSLOT)}
    dout = {}
    for i in range(NCHUNK):
        if i >= 1 and i + 2 < NCHUNK:
            dout[i - 1].wait()
            din[i + 2] = start_in(i + 2)
        din[i].wait()
        dout[i] = start_out(i)
    for i in range(NCHUNK - NSLOT, NCHUNK):
        dout[i].wait()


def _mul_body(a_ref, b_ref, o_ref):
    o_ref[...] = a_ref[...] * b_ref[...]


_tc_mul = pl.pallas_call(
    _mul_body,
    out_shape=jax.ShapeDtypeStruct((ROWS, 128), jnp.float32),
    grid=(ROWS // BLK,),
    in_specs=[
        pl.BlockSpec((BLK, 128), lambda i: (i, 0)),
        pl.BlockSpec((BLK, 128), lambda i: (i, 0)),
    ],
    out_specs=pl.BlockSpec((BLK, 128), lambda i: (i, 0)),
)


def kernel(m0, m1, m2):
    del m2
    pm0 = _to_phys(m0)
    pm1 = _to_phys(m1)
    out0 = _from_phys(_sc_copy(pm0))
    out1 = _from_phys(_tc_mul(pm0.reshape(ROWS, 128), pm1.reshape(ROWS, 128)).reshape(-1))
    return out0, out1


# speedup vs baseline: 24.1821x; 1.6693x over previous
import jax
import jax.numpy as jnp
from jax.experimental import pallas as pl

SHAPE = (512, 2, 300, 25)
NTOT = 512 * 2 * 300 * 25
ROWS = NTOT // 128
BLK = 6000


def _to_phys(x):
    return x.reshape(4, 128, 2, 300, 25).transpose(3, 4, 0, 2, 1).reshape(ROWS, 128)


def _from_phys(v):
    return v.reshape(300, 25, 4, 2, 128).transpose(2, 4, 3, 0, 1).reshape(SHAPE)


def _body(a_ref, b_ref, o0_ref, o1_ref):
    a = a_ref[...]
    o0_ref[...] = a
    o1_ref[...] = a * b_ref[...]


_mask_mul = pl.pallas_call(
    _body,
    out_shape=(
        jax.ShapeDtypeStruct((ROWS, 128), jnp.float32),
        jax.ShapeDtypeStruct((ROWS, 128), jnp.float32),
    ),
    grid=(ROWS // BLK,),
    in_specs=[
        pl.BlockSpec((BLK, 128), lambda i: (i, 0)),
        pl.BlockSpec((BLK, 128), lambda i: (i, 0)),
    ],
    out_specs=(
        pl.BlockSpec((BLK, 128), lambda i: (i, 0)),
        pl.BlockSpec((BLK, 128), lambda i: (i, 0)),
    ),
)


def kernel(m0, m1, m2):
    del m2
    out0, out1 = _mask_mul(_to_phys(m0), _to_phys(m1))
    return _from_phys(out0), _from_phys(out1)


# speedup vs baseline: 25.1487x; 1.0400x over previous
import jax
import jax.numpy as jnp
from jax.experimental import pallas as pl

SHAPE = (512, 2, 300, 25)
NTOT = 512 * 2 * 300 * 25
ROWS = NTOT // 128
BLK = 10000


def _to_phys(x):
    return x.reshape(4, 128, 2, 300, 25).transpose(3, 4, 0, 2, 1).reshape(ROWS, 128)


def _from_phys(v):
    return v.reshape(300, 25, 4, 2, 128).transpose(2, 4, 3, 0, 1).reshape(SHAPE)


def _body(a_ref, b_ref, o0_ref, o1_ref):
    a = a_ref[...]
    o0_ref[...] = a
    o1_ref[...] = a * b_ref[...]


_mask_mul = pl.pallas_call(
    _body,
    out_shape=(
        jax.ShapeDtypeStruct((ROWS, 128), jnp.float32),
        jax.ShapeDtypeStruct((ROWS, 128), jnp.float32),
    ),
    grid=(ROWS // BLK,),
    in_specs=[
        pl.BlockSpec((BLK, 128), lambda i: (i, 0)),
        pl.BlockSpec((BLK, 128), lambda i: (i, 0)),
    ],
    out_specs=(
        pl.BlockSpec((BLK, 128), lambda i: (i, 0)),
        pl.BlockSpec((BLK, 128), lambda i: (i, 0)),
    ),
)


def kernel(m0, m1, m2):
    del m2
    out0, out1 = _mask_mul(_to_phys(m0), _to_phys(m1))
    return _from_phys(out0), _from_phys(out1)


# speedup vs baseline: 25.2295x; 1.0032x over previous
import jax
import jax.numpy as jnp
from jax.experimental import pallas as pl

SHAPE = (512, 2, 300, 25)
NTOT = 512 * 2 * 300 * 25
ROWS = NTOT // 128
BLK = 12000


def _to_phys(x):
    return x.reshape(4, 128, 2, 300, 25).transpose(3, 4, 0, 2, 1).reshape(ROWS, 128)


def _from_phys(v):
    return v.reshape(300, 25, 4, 2, 128).transpose(2, 4, 3, 0, 1).reshape(SHAPE)


def _body(a_ref, b_ref, o0_ref, o1_ref):
    a = a_ref[...]
    o0_ref[...] = a
    o1_ref[...] = a * b_ref[...]


_mask_mul = pl.pallas_call(
    _body,
    out_shape=(
        jax.ShapeDtypeStruct((ROWS, 128), jnp.float32),
        jax.ShapeDtypeStruct((ROWS, 128), jnp.float32),
    ),
    grid=(ROWS // BLK,),
    in_specs=[
        pl.BlockSpec((BLK, 128), lambda i: (i, 0)),
        pl.BlockSpec((BLK, 128), lambda i: (i, 0)),
    ],
    out_specs=(
        pl.BlockSpec((BLK, 128), lambda i: (i, 0)),
        pl.BlockSpec((BLK, 128), lambda i: (i, 0)),
    ),
)


def kernel(m0, m1, m2):
    del m2
    out0, out1 = _mask_mul(_to_phys(m0), _to_phys(m1))
    return _from_phys(out0), _from_phys(out1)
